# Initial kernel scaffold; baseline (speedup 1.0000x reference)
import jax
import jax.numpy as jnp
from jax.experimental import pallas as pl


def kernel(node_features, edge_features, params, edge_index):
    raise NotImplementedError("write your pallas kernel here")


# speedup vs baseline: 2.5251x; 2.5251x over previous
import functools

import jax
import jax.numpy as jnp
from jax import lax
from jax.experimental import pallas as pl
from jax.experimental.pallas import tpu as pltpu
from jax.experimental.pallas import tpu_sc as plsc

F32 = jnp.float32

N = 10000
E = 320000
HID = 128
P = 15

NW = 32
GCH = 157
PER_W_G = GCH * 128
PAD2 = NW * PER_W_G
SCH = 125
SROW = 80
NPAD = 10240
TE = 2000
TN = 2000


def _ln_rows(o, g, b):
    m = jnp.mean(o, axis=-1, keepdims=True)
    v = jnp.mean((o - m) ** 2, axis=-1, keepdims=True)
    return (o - m) * lax.rsqrt(v + 1e-5) * g + b


def _edge_enc_body(x_ref, w1_ref, b1_ref, w2_ref, b2_ref, g_ref, bb_ref,
                   out_ref):
    h = jnp.maximum(
        jnp.dot(x_ref[...], w1_ref[...], preferred_element_type=F32)
        + b1_ref[...], 0.0)
    o = jnp.dot(h, w2_ref[...], preferred_element_type=F32) + b2_ref[...]
    out_ref[...] = _ln_rows(o, g_ref[...], bb_ref[...])


def _node_enc_body(x_ref, w1_ref, b1_ref, w2_ref, b2_ref, g_ref, bb_ref,
                   ws_ref, wd_ref, n_ref, t2_ref):
    h = jnp.maximum(
        jnp.dot(x_ref[...], w1_ref[...], preferred_element_type=F32)
        + b1_ref[...], 0.0)
    o = jnp.dot(h, w2_ref[...], preferred_element_type=F32) + b2_ref[...]
    nn = _ln_rows(o, g_ref[...], bb_ref[...])
    n_ref[...] = nn
    t2_ref[0] = jnp.dot(nn, ws_ref[...], preferred_element_type=F32)
    t2_ref[1] = jnp.dot(nn, wd_ref[...], preferred_element_type=F32)


def _edge_mlp_body(e_ref, gs_ref, gd_ref, w1_ref, b1_ref, w2_ref, b2_ref,
                   g_ref, bb_ref, out_ref):
    e = e_ref[...]
    pre = (jnp.dot(e, w1_ref[...], preferred_element_type=F32)
           + gs_ref[...] + gd_ref[...] + b1_ref[...])
    h = jnp.maximum(pre, 0.0)
    o = jnp.dot(h, w2_ref[...], preferred_element_type=F32) + b2_ref[...]
    out_ref[...] = _ln_rows(o, g_ref[...], bb_ref[...]) + e


def _node_blk_body(a0_ref, a1_ref, n_ref, wa_ref, wn_ref, b1_ref, w2_ref,
                   b2_ref, g_ref, bb_ref, ws_ref, wd_ref, n_out_ref,
                   t2_ref):
    n = n_ref[...]
    agg = a0_ref[0] + a1_ref[0]
    pre = (jnp.dot(agg, wa_ref[...], preferred_element_type=F32)
           + jnp.dot(n, wn_ref[...], preferred_element_type=F32)
           + b1_ref[...])
    h = jnp.maximum(pre, 0.0)
    o = jnp.dot(h, w2_ref[...], preferred_element_type=F32) + b2_ref[...]
    nn = _ln_rows(o, g_ref[...], bb_ref[...]) + n
    n_out_ref[...] = nn
    t2_ref[0] = jnp.dot(nn, ws_ref[...], preferred_element_type=F32)
    t2_ref[1] = jnp.dot(nn, wd_ref[...], preferred_element_type=F32)


def _node_blk_dec_body(a0_ref, a1_ref, n_ref, wa_ref, wn_ref, b1_ref,
                       w2_ref, b2_ref, g_ref, bb_ref, dw1_ref, db1_ref,
                       dw2_ref, db2_ref, dg_ref, db_ref, out_ref):
    n = n_ref[...]
    agg = a0_ref[0] + a1_ref[0]
    pre = (jnp.dot(agg, wa_ref[...], preferred_element_type=F32)
           + jnp.dot(n, wn_ref[...], preferred_element_type=F32)
           + b1_ref[...])
    h = jnp.maximum(pre, 0.0)
    o = jnp.dot(h, w2_ref[...], preferred_element_type=F32) + b2_ref[...]
    nn = _ln_rows(o, g_ref[...], bb_ref[...]) + n
    h2 = jnp.maximum(
        jnp.dot(nn, dw1_ref[...], preferred_element_type=F32)
        + db1_ref[...], 0.0)
    o2 = jnp.dot(h2, dw2_ref[...], preferred_element_type=F32) + db2_ref[...]
    lane = lax.broadcasted_iota(jnp.int32, o2.shape, 1)
    msk = lane < 3
    m = jnp.sum(jnp.where(msk, o2, 0.0), axis=-1, keepdims=True) / 3.0
    v = jnp.sum(jnp.where(msk, (o2 - m) ** 2, 0.0), axis=-1,
                keepdims=True) / 3.0
    out_ref[...] = (o2 - m) * lax.rsqrt(v + 1e-5) * dg_ref[...] + db_ref[...]


def _full(shape):
    return pl.BlockSpec(shape, lambda i: tuple(0 for _ in shape))


def _rows(bs):
    return pl.BlockSpec((bs, HID), lambda i: (i, 0))


def _edge_enc_call(x, p):
    return pl.pallas_call(
        _edge_enc_body,
        grid=(E // TE,),
        in_specs=[
            pl.BlockSpec((TE, 8), lambda i: (i, 0)),
            _full((8, HID)), _full((1, HID)), _full((HID, HID)),
            _full((1, HID)), _full((1, HID)), _full((1, HID)),
        ],
        out_specs=_rows(TE),
        out_shape=jax.ShapeDtypeStruct((E, HID), F32),
    )(x, p['w1'], p['b1'], p['w2'], p['b2'], p['ln_g'], p['ln_b'])


def _node_enc_call(x, p, ws, wd):
    return pl.pallas_call(
        _node_enc_body,
        grid=(N // TN,),
        in_specs=[
            _rows(TN),
            _full((HID, HID)), _full((1, HID)), _full((HID, HID)),
            _full((1, HID)), _full((1, HID)), _full((1, HID)),
            _full((HID, HID)), _full((HID, HID)),
        ],
        out_specs=[
            _rows(TN),
            pl.BlockSpec((2, TN, HID), lambda i: (0, i, 0)),
        ],
        out_shape=[
            jax.ShapeDtypeStruct((N, HID), F32),
            jax.ShapeDtypeStruct((2, N, HID), F32),
        ],
    )(x, p['w1'], p['b1'], p['w2'], p['b2'], p['ln_g'], p['ln_b'], ws, wd)


def _edge_mlp_call(e, g2, p):
    return pl.pallas_call(
        _edge_mlp_body,
        grid=(E // TE,),
        in_specs=[
            _rows(TE),
            pl.BlockSpec((TE, HID), lambda i: (i, 0)),
            pl.BlockSpec((TE, HID), lambda i: (E // TE + i, 0)),
            _full((HID, HID)), _full((1, HID)), _full((HID, HID)),
            _full((1, HID)), _full((1, HID)), _full((1, HID)),
        ],
        out_specs=_rows(TE),
        out_shape=jax.ShapeDtypeStruct((E, HID), F32),
    )(e, g2, g2, p['w1e'], p['b1'], p['w2'], p['b2'], p['ln_g'], p['ln_b'])


def _node_blk_call(parts, n, p, ws, wd):
    return pl.pallas_call(
        _node_blk_body,
        grid=(N // TN,),
        in_specs=[
            pl.BlockSpec((1, TN, HID), lambda i: (0, i, 0)),
            pl.BlockSpec((1, TN, HID), lambda i: (1, i, 0)),
            _rows(TN),
            _full((HID, HID)), _full((HID, HID)), _full((1, HID)),
            _full((HID, HID)), _full((1, HID)), _full((1, HID)),
            _full((1, HID)),
            _full((HID, HID)), _full((HID, HID)),
        ],
        out_specs=[
            _rows(TN),
            pl.BlockSpec((2, TN, HID), lambda i: (0, i, 0)),
        ],
        out_shape=[
            jax.ShapeDtypeStruct((N, HID), F32),
            jax.ShapeDtypeStruct((2, N, HID), F32),
        ],
    )(parts, parts, n, p['w1a'], p['w1n'], p['b1'], p['w2'], p['b2'],
      p['ln_g'], p['ln_b'], ws, wd)


def _node_blk_dec_call(parts, n, p, d):
    return pl.pallas_call(
        _node_blk_dec_body,
        grid=(N // TN,),
        in_specs=[
            pl.BlockSpec((1, TN, HID), lambda i: (0, i, 0)),
            pl.BlockSpec((1, TN, HID), lambda i: (1, i, 0)),
            _rows(TN),
            _full((HID, HID)), _full((HID, HID)), _full((1, HID)),
            _full((HID, HID)), _full((1, HID)), _full((1, HID)),
            _full((1, HID)),
            _full((HID, HID)), _full((1, HID)), _full((HID, HID)),
            _full((1, HID)), _full((1, HID)), _full((1, HID)),
        ],
        out_specs=_rows(TN),
        out_shape=jax.ShapeDtypeStruct((N, HID), F32),
    )(parts, parts, n, p['w1a'], p['w1n'], p['b1'], p['w2'], p['b2'],
      p['ln_g'], p['ln_b'], d['w1'], d['b1'], d['w2p'], d['b2p'],
      d['gp'], d['bp'])


_MESH = plsc.VectorSubcoreMesh(core_axis_name="c", subcore_axis_name="s")


@functools.partial(
    pl.kernel,
    out_type=jax.ShapeDtypeStruct((PAD2, HID), F32),
    mesh=_MESH,
    scratch_types=[
        pltpu.VMEM((GCH, 128), jnp.int32),
        pltpu.VMEM((128, HID), F32),
        pltpu.SemaphoreType.DMA,
    ],
)
def _sc_gather(table_hbm, idx_hbm, out_hbm, idx_v, rows_v, sem):
    wid = lax.axis_index("s") * 2 + lax.axis_index("c")
    base = wid * PER_W_G
    pltpu.sync_copy(idx_hbm.at[wid], idx_v)

    def body(j, carry):
        pltpu.async_copy(table_hbm.at[idx_v.at[j]], rows_v, sem).wait()
        pltpu.sync_copy(rows_v, out_hbm.at[pl.ds(base + j * 128, 128)])
        return carry

    lax.fori_loop(0, GCH, body, 0)


@functools.partial(
    pl.kernel,
    out_type=jax.ShapeDtypeStruct((2, NPAD, HID), F32),
    mesh=_MESH,
    scratch_types=[
        pltpu.VMEM((SCH, SROW), jnp.int32),
        pltpu.VMEM((SROW, HID), F32),
        pltpu.VMEM_SHARED((NPAD, HID), F32),
        pltpu.SemaphoreType.DMA,
    ],
)
def _sc_scatter(e_hbm, dst_hbm, zero_hbm, out_hbm, idx_v, rows_v, acc, sem):
    cid = lax.axis_index("c")
    sid = lax.axis_index("s")
    wid = sid * 2 + cid
    stripe = NPAD // 16
    pltpu.sync_copy(zero_hbm, acc.at[pl.ds(sid * stripe, stripe)])
    plsc.subcore_barrier()
    pltpu.sync_copy(dst_hbm.at[wid], idx_v)
    ebase = wid * (E // NW)

    def body(j, carry):
        pltpu.sync_copy(e_hbm.at[pl.ds(ebase + j * SROW, SROW)], rows_v)
        pltpu.sync_copy(rows_v, acc.at[idx_v.at[j]], add=True)
        return carry

    lax.fori_loop(0, SCH, body, 0)
    plsc.subcore_barrier()
    pltpu.sync_copy(acc.at[pl.ds(sid * stripe, stripe)],
                    out_hbm.at[cid, pl.ds(sid * stripe, stripe)])


def _prep_params(params):
    def r(b):
        return b.reshape(1, -1)

    pe = params['edge_enc']
    edge_enc = {
        'w1': jnp.pad(pe['w1'], ((0, 4), (0, 0))),
        'b1': r(pe['b1']), 'w2': pe['w2'], 'b2': r(pe['b2']),
        'ln_g': r(pe['ln_g']), 'ln_b': r(pe['ln_b']),
    }
    pn = params['node_enc']
    node_enc = {
        'w1': pn['w1'], 'b1': r(pn['b1']), 'w2': pn['w2'],
        'b2': r(pn['b2']), 'ln_g': r(pn['ln_g']), 'ln_b': r(pn['ln_b']),
    }
    eb = []
    for p in params['edge_blocks']:
        eb.append({
            'w1e': p['w1'][:HID], 'w1s': p['w1'][HID:2 * HID],
            'w1d': p['w1'][2 * HID:], 'b1': r(p['b1']), 'w2': p['w2'],
            'b2': r(p['b2']), 'ln_g': r(p['ln_g']), 'ln_b': r(p['ln_b']),
        })
    nb = []
    for p in params['node_blocks']:
        nb.append({
            'w1a': p['w1'][:HID], 'w1n': p['w1'][HID:], 'b1': r(p['b1']),
            'w2': p['w2'], 'b2': r(p['b2']), 'ln_g': r(p['ln_g']),
            'ln_b': r(p['ln_b']),
        })
    pd = params['decoder']
    dec = {
        'w1': pd['w1'], 'b1': r(pd['b1']),
        'w2p': jnp.pad(pd['w2'], ((0, 0), (0, HID - 3))),
        'b2p': r(jnp.pad(pd['b2'], (0, HID - 3))),
        'gp': r(jnp.pad(pd['ln_g'], (0, HID - 3))),
        'bp': r(jnp.pad(pd['ln_b'], (0, HID - 3))),
    }
    return edge_enc, node_enc, eb, nb, dec


def kernel(node_features, edge_features, params, edge_index):
    src = edge_index[0]
    dst = edge_index[1]
    edge_enc, node_enc, eb, nb, dec = _prep_params(params)

    idx2 = jnp.concatenate(
        [src, dst + N, jnp.zeros((PAD2 - 2 * E,), jnp.int32)]
    ).reshape(NW, GCH, 128)
    dst2 = dst.reshape(NW, SCH, SROW)
    zeros = jnp.zeros((NPAD // 16, HID), F32)
    x_e = jnp.pad(edge_features, ((0, 0), (0, 4)))

    e = _edge_enc_call(x_e, edge_enc)
    n, t2 = _node_enc_call(node_features, node_enc, eb[0]['w1s'],
                           eb[0]['w1d'])
    for i in range(P):
        g2 = _sc_gather(t2.reshape(2 * N, HID), idx2)
        e = _edge_mlp_call(e, g2, eb[i])
        parts = _sc_scatter(e, dst2, zeros)
        if i < P - 1:
            n, t2 = _node_blk_call(parts, n, nb[i], eb[i + 1]['w1s'],
                                   eb[i + 1]['w1d'])
        else:
            out = _node_blk_dec_call(parts, n, nb[i], dec)
    return out[:, :3]


# speedup vs baseline: 3.1215x; 1.2362x over previous
import functools

import jax
import jax.numpy as jnp
from jax import lax
from jax.experimental import pallas as pl
from jax.experimental.pallas import tpu as pltpu
from jax.experimental.pallas import tpu_sc as plsc

F32 = jnp.float32

N = 10000
E = 320000
HID = 128
P = 15

NW = 32
GCH = 157
PER_W_G = GCH * 128
PAD2 = NW * PER_W_G
SCH = 125
SROW = 80
NPAD = 10240
TE = 2000
TN = 2000


def _ln_rows(o, g, b):
    m = jnp.mean(o, axis=-1, keepdims=True)
    v = jnp.mean((o - m) ** 2, axis=-1, keepdims=True)
    return (o - m) * lax.rsqrt(v + 1e-5) * g + b


def _edge_enc_body(x_ref, w1_ref, b1_ref, w2_ref, b2_ref, g_ref, bb_ref,
                   out_ref):
    h = jnp.maximum(
        jnp.dot(x_ref[...], w1_ref[...], preferred_element_type=F32)
        + b1_ref[...], 0.0)
    o = jnp.dot(h, w2_ref[...], preferred_element_type=F32) + b2_ref[...]
    out_ref[...] = _ln_rows(o, g_ref[...], bb_ref[...])


def _node_enc_body(x_ref, w1_ref, b1_ref, w2_ref, b2_ref, g_ref, bb_ref,
                   ws_ref, wd_ref, n_ref, t2_ref):
    h = jnp.maximum(
        jnp.dot(x_ref[...], w1_ref[...], preferred_element_type=F32)
        + b1_ref[...], 0.0)
    o = jnp.dot(h, w2_ref[...], preferred_element_type=F32) + b2_ref[...]
    nn = _ln_rows(o, g_ref[...], bb_ref[...])
    n_ref[...] = nn
    t2_ref[0] = jnp.dot(nn, ws_ref[...], preferred_element_type=F32)
    t2_ref[1] = jnp.dot(nn, wd_ref[...], preferred_element_type=F32)


def _edge_mlp_body(e_ref, gs_ref, gd_ref, w1_ref, b1_ref, w2_ref, b2_ref,
                   g_ref, bb_ref, out_ref):
    e = e_ref[...]
    pre = (jnp.dot(e, w1_ref[...], preferred_element_type=F32)
           + gs_ref[...] + gd_ref[...] + b1_ref[...])
    h = jnp.maximum(pre, 0.0)
    o = jnp.dot(h, w2_ref[...], preferred_element_type=F32) + b2_ref[...]
    out_ref[...] = _ln_rows(o, g_ref[...], bb_ref[...]) + e


def _node_blk_body(a0_ref, a1_ref, n_ref, wa_ref, wn_ref, b1_ref, w2_ref,
                   b2_ref, g_ref, bb_ref, ws_ref, wd_ref, n_out_ref,
                   t2_ref):
    n = n_ref[...]
    agg = a0_ref[0] + a1_ref[0]
    pre = (jnp.dot(agg, wa_ref[...], preferred_element_type=F32)
           + jnp.dot(n, wn_ref[...], preferred_element_type=F32)
           + b1_ref[...])
    h = jnp.maximum(pre, 0.0)
    o = jnp.dot(h, w2_ref[...], preferred_element_type=F32) + b2_ref[...]
    nn = _ln_rows(o, g_ref[...], bb_ref[...]) + n
    n_out_ref[...] = nn
    t2_ref[0] = jnp.dot(nn, ws_ref[...], preferred_element_type=F32)
    t2_ref[1] = jnp.dot(nn, wd_ref[...], preferred_element_type=F32)


def _node_blk_dec_body(a0_ref, a1_ref, n_ref, wa_ref, wn_ref, b1_ref,
                       w2_ref, b2_ref, g_ref, bb_ref, dw1_ref, db1_ref,
                       dw2_ref, db2_ref, dg_ref, db_ref, out_ref):
    n = n_ref[...]
    agg = a0_ref[0] + a1_ref[0]
    pre = (jnp.dot(agg, wa_ref[...], preferred_element_type=F32)
           + jnp.dot(n, wn_ref[...], preferred_element_type=F32)
           + b1_ref[...])
    h = jnp.maximum(pre, 0.0)
    o = jnp.dot(h, w2_ref[...], preferred_element_type=F32) + b2_ref[...]
    nn = _ln_rows(o, g_ref[...], bb_ref[...]) + n
    h2 = jnp.maximum(
        jnp.dot(nn, dw1_ref[...], preferred_element_type=F32)
        + db1_ref[...], 0.0)
    o2 = jnp.dot(h2, dw2_ref[...], preferred_element_type=F32) + db2_ref[...]
    lane = lax.broadcasted_iota(jnp.int32, o2.shape, 1)
    msk = lane < 3
    m = jnp.sum(jnp.where(msk, o2, 0.0), axis=-1, keepdims=True) / 3.0
    v = jnp.sum(jnp.where(msk, (o2 - m) ** 2, 0.0), axis=-1,
                keepdims=True) / 3.0
    out_ref[...] = (o2 - m) * lax.rsqrt(v + 1e-5) * dg_ref[...] + db_ref[...]


def _full(shape):
    return pl.BlockSpec(shape, lambda i: tuple(0 for _ in shape))


def _rows(bs):
    return pl.BlockSpec((bs, HID), lambda i: (i, 0))


def _edge_enc_call(x, p):
    return pl.pallas_call(
        _edge_enc_body,
        grid=(E // TE,),
        in_specs=[
            pl.BlockSpec((TE, 8), lambda i: (i, 0)),
            _full((8, HID)), _full((1, HID)), _full((HID, HID)),
            _full((1, HID)), _full((1, HID)), _full((1, HID)),
        ],
        out_specs=_rows(TE),
        out_shape=jax.ShapeDtypeStruct((E, HID), F32),
    )(x, p['w1'], p['b1'], p['w2'], p['b2'], p['ln_g'], p['ln_b'])


def _node_enc_call(x, p, ws, wd):
    return pl.pallas_call(
        _node_enc_body,
        grid=(N // TN,),
        in_specs=[
            _rows(TN),
            _full((HID, HID)), _full((1, HID)), _full((HID, HID)),
            _full((1, HID)), _full((1, HID)), _full((1, HID)),
            _full((HID, HID)), _full((HID, HID)),
        ],
        out_specs=[
            _rows(TN),
            pl.BlockSpec((2, TN, HID), lambda i: (0, i, 0)),
        ],
        out_shape=[
            jax.ShapeDtypeStruct((N, HID), F32),
            jax.ShapeDtypeStruct((2, N, HID), F32),
        ],
    )(x, p['w1'], p['b1'], p['w2'], p['b2'], p['ln_g'], p['ln_b'], ws, wd)


def _edge_mlp_call(e, g2, p):
    return pl.pallas_call(
        _edge_mlp_body,
        grid=(E // TE,),
        in_specs=[
            _rows(TE),
            pl.BlockSpec((TE, HID), lambda i: (i, 0)),
            pl.BlockSpec((TE, HID), lambda i: (E // TE + i, 0)),
            _full((HID, HID)), _full((1, HID)), _full((HID, HID)),
            _full((1, HID)), _full((1, HID)), _full((1, HID)),
        ],
        out_specs=_rows(TE),
        out_shape=jax.ShapeDtypeStruct((E, HID), F32),
    )(e, g2, g2, p['w1e'], p['b1'], p['w2'], p['b2'], p['ln_g'], p['ln_b'])


def _node_blk_call(parts, n, p, ws, wd):
    return pl.pallas_call(
        _node_blk_body,
        grid=(N // TN,),
        in_specs=[
            pl.BlockSpec((1, TN, HID), lambda i: (0, i, 0)),
            pl.BlockSpec((1, TN, HID), lambda i: (1, i, 0)),
            _rows(TN),
            _full((HID, HID)), _full((HID, HID)), _full((1, HID)),
            _full((HID, HID)), _full((1, HID)), _full((1, HID)),
            _full((1, HID)),
            _full((HID, HID)), _full((HID, HID)),
        ],
        out_specs=[
            _rows(TN),
            pl.BlockSpec((2, TN, HID), lambda i: (0, i, 0)),
        ],
        out_shape=[
            jax.ShapeDtypeStruct((N, HID), F32),
            jax.ShapeDtypeStruct((2, N, HID), F32),
        ],
    )(parts, parts, n, p['w1a'], p['w1n'], p['b1'], p['w2'], p['b2'],
      p['ln_g'], p['ln_b'], ws, wd)


def _node_blk_dec_call(parts, n, p, d):
    return pl.pallas_call(
        _node_blk_dec_body,
        grid=(N // TN,),
        in_specs=[
            pl.BlockSpec((1, TN, HID), lambda i: (0, i, 0)),
            pl.BlockSpec((1, TN, HID), lambda i: (1, i, 0)),
            _rows(TN),
            _full((HID, HID)), _full((HID, HID)), _full((1, HID)),
            _full((HID, HID)), _full((1, HID)), _full((1, HID)),
            _full((1, HID)),
            _full((HID, HID)), _full((1, HID)), _full((HID, HID)),
            _full((1, HID)), _full((1, HID)), _full((1, HID)),
        ],
        out_specs=_rows(TN),
        out_shape=jax.ShapeDtypeStruct((N, HID), F32),
    )(parts, parts, n, p['w1a'], p['w1n'], p['b1'], p['w2'], p['b2'],
      p['ln_g'], p['ln_b'], d['w1'], d['b1'], d['w2p'], d['b2p'],
      d['gp'], d['bp'])


_MESH = plsc.VectorSubcoreMesh(core_axis_name="c", subcore_axis_name="s")


@functools.partial(
    pl.kernel,
    out_type=jax.ShapeDtypeStruct((PAD2, HID), F32),
    mesh=_MESH,
    scratch_types=[
        pltpu.VMEM((GCH, 128), jnp.int32),
        pltpu.VMEM((2, 128, HID), F32),
        pltpu.SemaphoreType.DMA((2,)),
    ],
)
def _sc_gather(table_hbm, idx_hbm, out_hbm, idx_v, rows_v, sems):
    wid = lax.axis_index("s") * 2 + lax.axis_index("c")
    base = wid * PER_W_G
    pltpu.sync_copy(idx_hbm.at[wid], idx_v)

    def start(j, p):
        pltpu.async_copy(table_hbm.at[idx_v.at[j]], rows_v.at[p],
                         sems.at[p])

    start(0, 0)

    def body(j, carry):
        p = j % 2

        @pl.when(j + 1 < GCH)
        def _():
            start(j + 1, 1 - p)

        pltpu.make_async_copy(out_hbm.at[pl.ds(base, 128)], rows_v.at[p],
                              sems.at[p]).wait()
        pltpu.sync_copy(rows_v.at[p], out_hbm.at[pl.ds(base + j * 128, 128)])
        return carry

    lax.fori_loop(0, GCH, body, 0)


@functools.partial(
    pl.kernel,
    out_type=jax.ShapeDtypeStruct((2, NPAD, HID), F32),
    mesh=_MESH,
    scratch_types=[
        pltpu.VMEM((SCH, SROW), jnp.int32),
        pltpu.VMEM((2, SROW, HID), F32),
        pltpu.VMEM_SHARED((NPAD, HID), F32),
        pltpu.SemaphoreType.DMA((2,)),
    ],
)
def _sc_scatter(e_hbm, dst_hbm, zero_hbm, out_hbm, idx_v, rows_v, acc, sems):
    cid = lax.axis_index("c")
    sid = lax.axis_index("s")
    wid = sid * 2 + cid
    stripe = NPAD // 16
    pltpu.sync_copy(zero_hbm, acc.at[pl.ds(sid * stripe, stripe)])
    plsc.subcore_barrier()
    pltpu.sync_copy(dst_hbm.at[wid], idx_v)
    ebase = wid * (E // NW)

    def start(j, p):
        pltpu.async_copy(e_hbm.at[pl.ds(ebase + j * SROW, SROW)],
                         rows_v.at[p], sems.at[p])

    start(0, 0)

    def body(j, carry):
        p = j % 2

        @pl.when(j + 1 < SCH)
        def _():
            start(j + 1, 1 - p)

        pltpu.make_async_copy(e_hbm.at[pl.ds(ebase, SROW)], rows_v.at[p],
                              sems.at[p]).wait()
        pltpu.sync_copy(rows_v.at[p], acc.at[idx_v.at[j]], add=True)
        return carry

    lax.fori_loop(0, SCH, body, 0)
    plsc.subcore_barrier()
    pltpu.sync_copy(acc.at[pl.ds(sid * stripe, stripe)],
                    out_hbm.at[cid, pl.ds(sid * stripe, stripe)])


def _prep_params(params):
    def r(b):
        return b.reshape(1, -1)

    pe = params['edge_enc']
    edge_enc = {
        'w1': jnp.pad(pe['w1'], ((0, 4), (0, 0))),
        'b1': r(pe['b1']), 'w2': pe['w2'], 'b2': r(pe['b2']),
        'ln_g': r(pe['ln_g']), 'ln_b': r(pe['ln_b']),
    }
    pn = params['node_enc']
    node_enc = {
        'w1': pn['w1'], 'b1': r(pn['b1']), 'w2': pn['w2'],
        'b2': r(pn['b2']), 'ln_g': r(pn['ln_g']), 'ln_b': r(pn['ln_b']),
    }
    eb = []
    for p in params['edge_blocks']:
        eb.append({
            'w1e': p['w1'][:HID], 'w1s': p['w1'][HID:2 * HID],
            'w1d': p['w1'][2 * HID:], 'b1': r(p['b1']), 'w2': p['w2'],
            'b2': r(p['b2']), 'ln_g': r(p['ln_g']), 'ln_b': r(p['ln_b']),
        })
    nb = []
    for p in params['node_blocks']:
        nb.append({
            'w1a': p['w1'][:HID], 'w1n': p['w1'][HID:], 'b1': r(p['b1']),
            'w2': p['w2'], 'b2': r(p['b2']), 'ln_g': r(p['ln_g']),
            'ln_b': r(p['ln_b']),
        })
    pd = params['decoder']
    dec = {
        'w1': pd['w1'], 'b1': r(pd['b1']),
        'w2p': jnp.pad(pd['w2'], ((0, 0), (0, HID - 3))),
        'b2p': r(jnp.pad(pd['b2'], (0, HID - 3))),
        'gp': r(jnp.pad(pd['ln_g'], (0, HID - 3))),
        'bp': r(jnp.pad(pd['ln_b'], (0, HID - 3))),
    }
    return edge_enc, node_enc, eb, nb, dec


def kernel(node_features, edge_features, params, edge_index):
    src = edge_index[0]
    dst = edge_index[1]
    edge_enc, node_enc, eb, nb, dec = _prep_params(params)

    idx2 = jnp.concatenate(
        [src, dst + N, jnp.zeros((PAD2 - 2 * E,), jnp.int32)]
    ).reshape(NW, GCH, 128)
    dst2 = dst.reshape(NW, SCH, SROW)
    zeros = jnp.zeros((NPAD // 16, HID), F32)
    x_e = jnp.pad(edge_features, ((0, 0), (0, 4)))

    e = _edge_enc_call(x_e, edge_enc)
    n, t2 = _node_enc_call(node_features, node_enc, eb[0]['w1s'],
                           eb[0]['w1d'])
    for i in range(P):
        g2 = _sc_gather(t2.reshape(2 * N, HID), idx2)
        e = _edge_mlp_call(e, g2, eb[i])
        parts = _sc_scatter(e, dst2, zeros)
        if i < P - 1:
            n, t2 = _node_blk_call(parts, n, nb[i], eb[i + 1]['w1s'],
                                   eb[i + 1]['w1d'])
        else:
            out = _node_blk_dec_call(parts, n, nb[i], dec)
    return out[:, :3]
